# baseline (speedup 1.0000x reference)
import jax
import jax.numpy as jnp
from jax import lax
from jax.experimental import pallas as pl
from jax.experimental.pallas import tpu as pltpu
from jax.experimental.pallas import tpu_sc as plsc

DIM = 64
FIELDS = 26
SKEWW = 144

NC = 2
NS = 16
NW = NC * NS

BT = 128
BT_PER_W = 4
NBLK = FIELDS * BT_PER_W


def _body(xT_hbm, table_hbm, out_hbm, idx_v, rows0, rows1, skew, st0, st1,
          gsem0, gsem1, wsem0, wsem1):
    wid = lax.axis_index("s") * NC + lax.axis_index("c")
    b0 = wid * (BT_PER_W * BT)

    pltpu.sync_copy(xT_hbm.at[:, pl.ds(b0, BT_PER_W * BT)], idx_v)

    rows = (rows0, rows1)
    st = (st0, st1)
    gsems = (gsem0, gsem1)
    wsems = (wsem0, wsem1)

    def fire_gather(f, btl, buf, sem):
        pltpu.async_copy(
            table_hbm.at[idx_v.at[f, pl.ds(btl * BT, BT)]], buf, sem)

    def wait_gather(buf, sem):
        pltpu.make_async_copy(table_hbm.at[pl.ds(0, BT)], buf, sem).wait()

    def fire_write(f, btl, buf, sem):
        bt = wid * BT_PER_W + btl
        for db in range(8):
            pltpu.async_copy(buf.at[pl.ds(db * 8, 8)], out_hbm.at[f, db, bt], sem)

    def wait_write(buf, sem):
        d = pltpu.make_async_copy(out_hbm.at[0, 0, 0], buf.at[pl.ds(0, 8)], sem)
        for _ in range(8):
            d.wait()

    iota = lax.iota(jnp.int32, 16)
    cvecs = [iota + c0 for c0 in range(0, DIM, 16)]

    def transpose(rbuf, sbuf):
        for b in range(BT):
            vb = iota + b
            for cg in range(DIM // 16):
                vals = rbuf[b, pl.ds(cg * 16, 16)]
                plsc.store_scatter(skew, [cvecs[cg], vb], vals)
        for d in range(DIM):
            off = d % 16
            for j in range(BT // 16):
                sbuf[d, pl.ds(j * 16, 16)] = skew[d, pl.ds(off + j * 16, 16)]

    def advance(f, btl):
        wrap = f == FIELDS - 1
        return (jnp.where(wrap, 0, f + 1),
                jnp.where(wrap, btl + 1, btl))

    fire_gather(0, 0, rows0, gsem0)
    fire_gather(1, 0, rows1, gsem1)

    def body(i, carry):
        f, btl = carry
        for b in range(2):
            k = i * 2 + b
            f1, btl1 = advance(f, btl)
            f2, btl2 = advance(f1, btl1)

            @pl.when(k >= 2)
            def _():
                wait_write(st[b], wsems[b])

            wait_gather(rows[b], gsems[b])
            transpose(rows[b], st[b])
            fire_write(f, btl, st[b], wsems[b])

            @pl.when(k <= NBLK - 3)
            def _():
                fire_gather(f2, btl2, rows[b], gsems[b])

            f, btl = f1, btl1
        return (f, btl)

    lax.fori_loop(0, NBLK // 2, body, (jnp.int32(0), jnp.int32(0)))

    for b in range(2):
        wait_write(st[b], wsems[b])


def kernel(x, table):
    batch, fields = x.shape
    xT = jnp.transpose(x)

    out5 = pl.kernel(
        _body,
        out_type=jax.ShapeDtypeStruct(
            (FIELDS, DIM // 8, batch // BT, 8, BT), jnp.float32),
        mesh=plsc.VectorSubcoreMesh(core_axis_name="c", subcore_axis_name="s"),
        compiler_params=pltpu.CompilerParams(
            use_tc_tiling_on_sc=False, needs_layout_passes=False),
        scratch_types=[
            pltpu.VMEM((FIELDS, BT_PER_W * BT), jnp.int32),
            pltpu.VMEM((BT, DIM), jnp.float32),
            pltpu.VMEM((BT, DIM), jnp.float32),
            pltpu.VMEM((DIM, SKEWW), jnp.float32),
            pltpu.VMEM((DIM, BT), jnp.float32),
            pltpu.VMEM((DIM, BT), jnp.float32),
            pltpu.SemaphoreType.DMA,
            pltpu.SemaphoreType.DMA,
            pltpu.SemaphoreType.DMA,
            pltpu.SemaphoreType.DMA,
        ],
    )(xT, table)
    return jnp.transpose(out5, (2, 4, 0, 1, 3)).reshape(batch, fields, DIM)


# speedup vs baseline: 1.0060x; 1.0060x over previous
import jax
import jax.numpy as jnp
from jax import lax
from jax.experimental import pallas as pl
from jax.experimental.pallas import tpu as pltpu
from jax.experimental.pallas import tpu_sc as plsc

DIM = 64
FIELDS = 26
SKEWW = 144

NC = 2
NS = 16
NW = NC * NS

BT = 128
BT_PER_W = 4
NBLK = FIELDS * BT_PER_W


def _body(xT_hbm, table_hbm, out_hbm, idx_v, rows0, rows1, skew, st0, st1,
          gsem0, gsem1, wsem0, wsem1):
    wid = lax.axis_index("s") * NC + lax.axis_index("c")
    b0 = wid * (BT_PER_W * BT)

    pltpu.sync_copy(xT_hbm.at[:, pl.ds(b0, BT_PER_W * BT)], idx_v)

    rows = (rows0, rows1)
    st = (st0, st1)
    gsems = (gsem0, gsem1)
    wsems = (wsem0, wsem1)

    def fire_gather(f, btl, buf, sem):
        pltpu.async_copy(
            table_hbm.at[idx_v.at[f, pl.ds(btl * BT, BT)]], buf, sem)

    def wait_gather(buf, sem):
        pltpu.make_async_copy(table_hbm.at[pl.ds(0, BT)], buf, sem).wait()

    def fire_write(f, btl, buf, sem):
        bt = wid * BT_PER_W + btl
        for db in range(8):
            pltpu.async_copy(buf.at[db], out_hbm.at[f, db, bt], sem)

    def wait_write(buf, sem):
        pltpu.make_async_copy(out_hbm.at[0, :, 0], buf, sem).wait()

    iota = lax.iota(jnp.int32, 16)
    cbases = [(iota + c0) * SKEWW for c0 in range(0, DIM, 16)]

    def transpose(rbuf, sbuf):
        for b in range(BT):
            vb = iota + b
            for cg in range(DIM // 16):
                vals = rbuf[b, pl.ds(cg * 16, 16)]
                plsc.store_scatter(skew, [cbases[cg] + vb], vals)

        nj = BT // 16
        lead = 2

        def load_row(d):
            base = d * SKEWW + d % 16
            return [skew[pl.ds(base + j * 16, 16)] for j in range(nj)]

        pending = [load_row(d) for d in range(lead)]
        for d in range(DIM):
            if d + lead < DIM:
                pending.append(load_row(d + lead))
            vals = pending[d]
            for j in range(nj):
                sbuf[d // 8, d % 8, pl.ds(j * 16, 16)] = vals[j]

    def advance(f, btl):
        wrap = f == FIELDS - 1
        return (jnp.where(wrap, 0, f + 1),
                jnp.where(wrap, btl + 1, btl))

    fire_gather(0, 0, rows0, gsem0)
    fire_gather(1, 0, rows1, gsem1)

    def body(i, carry):
        f, btl = carry
        for b in range(2):
            k = i * 2 + b
            f1, btl1 = advance(f, btl)
            f2, btl2 = advance(f1, btl1)

            @pl.when(k >= 2)
            def _():
                wait_write(st[b], wsems[b])

            wait_gather(rows[b], gsems[b])
            transpose(rows[b], st[b])
            fire_write(f, btl, st[b], wsems[b])

            @pl.when(k <= NBLK - 3)
            def _():
                fire_gather(f2, btl2, rows[b], gsems[b])

            f, btl = f1, btl1
        return (f, btl)

    lax.fori_loop(0, NBLK // 2, body, (jnp.int32(0), jnp.int32(0)))

    for b in range(2):
        wait_write(st[b], wsems[b])


def kernel(x, table):
    batch, fields = x.shape
    xT = jnp.transpose(x)

    out5 = pl.kernel(
        _body,
        out_type=jax.ShapeDtypeStruct(
            (FIELDS, DIM // 8, batch // BT, 8, BT), jnp.float32),
        mesh=plsc.VectorSubcoreMesh(core_axis_name="c", subcore_axis_name="s"),
        compiler_params=pltpu.CompilerParams(
            use_tc_tiling_on_sc=False, needs_layout_passes=False),
        scratch_types=[
            pltpu.VMEM((FIELDS, BT_PER_W * BT), jnp.int32),
            pltpu.VMEM((BT, DIM), jnp.float32),
            pltpu.VMEM((BT, DIM), jnp.float32),
            pltpu.VMEM((DIM * SKEWW,), jnp.float32),
            pltpu.VMEM((8, 8, BT), jnp.float32),
            pltpu.VMEM((8, 8, BT), jnp.float32),
            pltpu.SemaphoreType.DMA,
            pltpu.SemaphoreType.DMA,
            pltpu.SemaphoreType.DMA,
            pltpu.SemaphoreType.DMA,
        ],
    )(xT, table)
    return jnp.transpose(out5, (2, 4, 0, 1, 3)).reshape(batch, fields, DIM)


# speedup vs baseline: 1.0495x; 1.0433x over previous
import jax
import jax.numpy as jnp
from jax import lax
from jax.experimental import pallas as pl
from jax.experimental.pallas import tpu as pltpu
from jax.experimental.pallas import tpu_sc as plsc

DIM = 64

NC = 2
NS = 16
NW = NC * NS

IDXW = 128
WIDE = 128
NBUF = 4


def _gather_body(idx_hbm, table_hbm, out_hbm, idx_v,
                 b0, b1, b2, b3, g0, g1, g2, g3, w0, w1, w2, w3):
    wid = lax.axis_index("s") * NC + lax.axis_index("c")
    n_chunks = idx_hbm.shape[1]
    base = wid * n_chunks * IDXW

    pltpu.sync_copy(idx_hbm.at[wid], idx_v)

    bufs = (b0, b1, b2, b3)
    gsems = (g0, g1, g2, g3)
    wsems = (w0, w1, w2, w3)

    def fire(c, buf, sem):
        pltpu.async_copy(table_hbm.at[idx_v.at[c]], buf, sem)

    def drain_gather(buf, sem):
        pltpu.make_async_copy(table_hbm.at[pl.ds(0, IDXW)], buf, sem).wait()

    def fire_write(c, buf, sem):
        pltpu.async_copy(buf.at[:, pl.ds(0, DIM)],
                         out_hbm.at[pl.ds(base + c * IDXW, IDXW)], sem)

    def drain_write(buf, sem):
        pltpu.make_async_copy(out_hbm.at[pl.ds(0, IDXW)],
                              buf.at[:, pl.ds(0, DIM)], sem).wait()

    for b in range(NBUF):
        fire(b, bufs[b], gsems[b])

    def body(i, carry):
        for b in range(NBUF):
            c = i * NBUF + b

            @pl.when(c >= NBUF)
            def _():
                drain_write(bufs[b], wsems[b])

            drain_gather(bufs[b], gsems[b])
            fire_write(c, bufs[b], wsems[b])

            @pl.when(c <= n_chunks - NBUF - 1)
            def _():
                fire(c + NBUF, bufs[b], gsems[b])

        return carry

    lax.fori_loop(0, n_chunks // NBUF, body, 0)

    for b in range(NBUF):
        drain_write(bufs[b], wsems[b])


def kernel(x, table):
    batch, fields = x.shape
    total = batch * fields
    rows_per_w = total // NW
    idx3 = x.reshape(NW, rows_per_w // IDXW, IDXW)
    table_wide = jnp.pad(table, ((0, 0), (0, WIDE - DIM)))

    out = pl.kernel(
        _gather_body,
        out_type=jax.ShapeDtypeStruct((total, DIM), jnp.float32),
        mesh=plsc.VectorSubcoreMesh(core_axis_name="c", subcore_axis_name="s"),
        compiler_params=pltpu.CompilerParams(use_tc_tiling_on_sc=False),
        scratch_types=(
            [pltpu.VMEM((rows_per_w // IDXW, IDXW), jnp.int32)]
            + [pltpu.VMEM((IDXW, WIDE), jnp.float32)] * NBUF
            + [pltpu.SemaphoreType.DMA] * (2 * NBUF)
        ),
    )(idx3, table_wide)
    return out.reshape(batch, fields, DIM)
